# Initial kernel scaffold; baseline (speedup 1.0000x reference)
import jax
import jax.numpy as jnp
from jax.experimental import pallas as pl


def kernel(x, slct, W, b):
    raise NotImplementedError("write your pallas kernel here")


# speedup vs baseline: 1.3889x; 1.3889x over previous
import jax
import jax.numpy as jnp
from jax.experimental import pallas as pl
from jax.experimental.pallas import tpu as pltpu

_B, _NIN, _NHID, _E = 2048, 1024, 1024, 8
_TILE = 128
_NT = 24
_C = _NT * _TILE


def _mm_body(te_ref, x_ref, w_ref, b_ref, o_ref):
    y = jax.lax.dot_general(
        x_ref[...], w_ref[0],
        dimension_numbers=(((1,), (1,)), ((), ())),
        preferred_element_type=jnp.float32)
    o_ref[...] = jnp.maximum(y + b_ref[0], 0.0)


def _expert_matmul(x_sorted, tile_expert, W, b):
    grid_spec = pltpu.PrefetchScalarGridSpec(
        num_scalar_prefetch=1,
        grid=(_NT,),
        in_specs=[
            pl.BlockSpec((_TILE, _NIN), lambda t, te: (t, 0)),
            pl.BlockSpec((1, _NHID, _NIN), lambda t, te: (te[t], 0, 0)),
            pl.BlockSpec((1, 1, _NHID), lambda t, te: (te[t], 0, 0)),
        ],
        out_specs=pl.BlockSpec((_TILE, _NHID), lambda t, te: (t, 0)),
    )
    return pl.pallas_call(
        _mm_body,
        grid_spec=grid_spec,
        out_shape=jax.ShapeDtypeStruct((_C, _NHID), jnp.float32),
    )(tile_expert, x_sorted, W, b.reshape(_E, 1, _NHID))


def kernel(x, slct, W, b):
    idx = jnp.argmax(slct, axis=1).astype(jnp.int32)
    oh = (idx[:, None] == jnp.arange(_E, dtype=jnp.int32)[None, :]).astype(jnp.int32)
    counts = jnp.sum(oh, axis=0)
    padded = ((counts + _TILE - 1) // _TILE) * _TILE
    ends = jnp.cumsum(padded)
    offs = ends - padded
    ranks = jnp.cumsum(oh, axis=0) - oh
    rank = jnp.sum(ranks * oh, axis=1)
    slot = offs[idx] + rank
    x_sorted = jnp.zeros((_C, _NIN), x.dtype).at[slot].set(x)
    tstart = jnp.arange(_NT, dtype=jnp.int32) * _TILE
    tile_expert = jnp.minimum(
        jnp.sum((tstart[:, None] >= ends[None, :]).astype(jnp.int32), axis=1),
        _E - 1).astype(jnp.int32)
    y_sorted = _expert_matmul(x_sorted, tile_expert, W, b)
    return y_sorted[slot]
